# initial kernel scaffold (unmeasured)
import jax
import jax.numpy as jnp
from jax import lax
from jax.experimental import pallas as pl
from jax.experimental.pallas import tpu as pltpu


def kernel(
    x,
):
    def body(*refs):
        pass

    out_shape = jax.ShapeDtypeStruct(..., jnp.float32)
    return pl.pallas_call(body, out_shape=out_shape)(...)


# baseline (device time: 48774 ns/iter reference)
import jax
import jax.numpy as jnp
from jax import lax
from jax.experimental import pallas as pl
from jax.experimental.pallas import tpu as pltpu

N_Z = 4
MESH = pl.DeviceIdType.MESH


def kernel(x):
    m_per, n = x.shape
    qrows = m_per // 4
    half = n // 2

    def body(x_ref, o_ref, zrs, zrr, zls, zlr,
             xs_, xr_, ys_, yr_, fxs, fxr, fys, fyr):
        cx = lax.axis_index("x")
        cy = lax.axis_index("y")
        z = lax.axis_index("z")
        q_me = 2 * cx + cy
        q_x = 2 * (1 - cx) + cy
        q_y = 2 * cx + (1 - cy)
        q_d = 2 * (1 - cx) + (1 - cy)
        xn = (1 - cx, cy, z)
        yn = (cx, 1 - cy, z)

        def cl(v):
            return jnp.clip(v, 0, N_Z - 1)

        def slab(j, q):
            return o_ref.at[pl.ds(j * m_per + q * qrows, qrows), :]

        def slab_half(j, q, hf):
            return o_ref.at[
                pl.ds(j * m_per + q * qrows, qrows), pl.ds(hf * half, half)
            ]

        bar = pltpu.get_barrier_semaphore()
        pl.semaphore_signal(bar, inc=1, device_id=xn, device_id_type=MESH)
        pl.semaphore_signal(bar, inc=1, device_id=yn, device_id_type=MESH)

        @pl.when(z >= 1)
        def _():
            pl.semaphore_signal(
                bar, inc=1, device_id=(cx, cy, cl(z - 1)), device_id_type=MESH
            )

        @pl.when(z <= 2)
        def _():
            pl.semaphore_signal(
                bar, inc=1, device_id=(cx, cy, cl(z + 1)), device_id_type=MESH
            )

        @pl.when(jnp.logical_and(z >= 1, z <= 2))
        def _():
            pl.semaphore_wait(bar, 4)

        @pl.when(jnp.logical_or(z == 0, z == 3))
        def _():
            pl.semaphore_wait(bar, 3)

        o_ref[pl.ds(z * m_per, m_per), :] = x_ref[...]

        drains = []

        def z_send(h, right):
            if right:
                cond = jnp.logical_and(z >= h, z <= 2)
                j, tgt, ss, rs = cl(z - h), (cx, cy, cl(z + 1)), zrs, zrr
            else:
                cond = jnp.logical_and(z >= 1, z + h <= 3)
                j, tgt, ss, rs = cl(z + h), (cx, cy, cl(z - 1)), zls, zlr
            d = pltpu.make_async_remote_copy(
                src_ref=slab(j, q_me), dst_ref=slab(j, q_me),
                send_sem=ss.at[h], recv_sem=rs.at[h],
                device_id=tgt, device_id_type=MESH,
            )

            @pl.when(cond)
            def _():
                d.start()

            drains.append((cond, d))

        def arr(h, right):
            if right:
                return z >= h + 1, cl(z - 1 - h)
            return z <= 2 - h, cl(z + 1 + h)

        def z_recv_wait(h, right):
            cond, j = arr(h, right)
            rs = zrr if right else zlr
            d = pltpu.make_async_remote_copy(
                src_ref=slab(j, q_me), dst_ref=slab(j, q_me),
                send_sem=rs.at[h], recv_sem=rs.at[h],
                device_id=(cx, cy, z), device_id_type=MESH,
            )

            @pl.when(cond)
            def _():
                d.wait_recv()

        def plane_direct_start(h, f):
            cond, j = arr(h, f)
            for sem_s, sem_r, tgt in ((xs_, xr_, xn), (ys_, yr_, yn)):
                d = pltpu.make_async_remote_copy(
                    src_ref=slab(j, q_me), dst_ref=slab(j, q_me),
                    send_sem=sem_s.at[int(f), h], recv_sem=sem_r.at[int(f), h],
                    device_id=tgt, device_id_type=MESH,
                )

                @pl.when(cond)
                def _():
                    d.start()

                drains.append((cond, d))

        def plane_recv_and_forward(h, f):
            cond, j = arr(h, f)
            dx = pltpu.make_async_remote_copy(
                src_ref=slab(j, q_x), dst_ref=slab(j, q_x),
                send_sem=xr_.at[int(f), h], recv_sem=xr_.at[int(f), h],
                device_id=(cx, cy, z), device_id_type=MESH,
            )
            dy = pltpu.make_async_remote_copy(
                src_ref=slab(j, q_y), dst_ref=slab(j, q_y),
                send_sem=yr_.at[int(f), h], recv_sem=yr_.at[int(f), h],
                device_id=(cx, cy, z), device_id_type=MESH,
            )
            fy = pltpu.make_async_remote_copy(
                src_ref=slab_half(j, q_x, 0), dst_ref=slab_half(j, q_x, 0),
                send_sem=fys.at[int(f), h], recv_sem=fyr.at[int(f), h],
                device_id=yn, device_id_type=MESH,
            )
            fx = pltpu.make_async_remote_copy(
                src_ref=slab_half(j, q_y, 1), dst_ref=slab_half(j, q_y, 1),
                send_sem=fxs.at[int(f), h], recv_sem=fxr.at[int(f), h],
                device_id=xn, device_id_type=MESH,
            )

            @pl.when(cond)
            def _():
                dx.wait_recv()
                fy.start()
                dy.wait_recv()
                fx.start()

            drains.append((cond, fy))
            drains.append((cond, fx))

        def fwd_recv_wait(h, f):
            cond, j = arr(h, f)
            da = pltpu.make_async_remote_copy(
                src_ref=slab_half(j, q_d, 0), dst_ref=slab_half(j, q_d, 0),
                send_sem=fys.at[int(f), h], recv_sem=fyr.at[int(f), h],
                device_id=(cx, cy, z), device_id_type=MESH,
            )
            db = pltpu.make_async_remote_copy(
                src_ref=slab_half(j, q_d, 1), dst_ref=slab_half(j, q_d, 1),
                send_sem=fxs.at[int(f), h], recv_sem=fxr.at[int(f), h],
                device_id=(cx, cy, z), device_id_type=MESH,
            )

            @pl.when(cond)
            def _():
                da.wait_recv()
                db.wait_recv()

        z_send(0, True)
        z_send(0, False)
        for h in range(N_Z - 1):
            z_recv_wait(h, True)
            z_recv_wait(h, False)
            if h < N_Z - 2:
                z_send(h + 1, True)
                z_send(h + 1, False)
            for f in (True, False):
                plane_direct_start(h, f)
            for f in (True, False):
                plane_recv_and_forward(h, f)
        for h in range(N_Z - 1):
            for f in (True, False):
                fwd_recv_wait(h, f)
        for cond, d in drains:
            @pl.when(cond)
            def _():
                d.wait_send()

    out_shape = jax.ShapeDtypeStruct((N_Z * m_per, n), jnp.float32)
    return pl.pallas_call(
        body,
        out_shape=out_shape,
        in_specs=[pl.BlockSpec(memory_space=pltpu.VMEM)],
        out_specs=pl.BlockSpec(memory_space=pltpu.VMEM),
        scratch_shapes=[
            pltpu.SemaphoreType.DMA((N_Z - 1,)),
            pltpu.SemaphoreType.DMA((N_Z - 1,)),
            pltpu.SemaphoreType.DMA((N_Z - 1,)),
            pltpu.SemaphoreType.DMA((N_Z - 1,)),
            pltpu.SemaphoreType.DMA((2, N_Z - 1)),
            pltpu.SemaphoreType.DMA((2, N_Z - 1)),
            pltpu.SemaphoreType.DMA((2, N_Z - 1)),
            pltpu.SemaphoreType.DMA((2, N_Z - 1)),
            pltpu.SemaphoreType.DMA((2, N_Z - 1)),
            pltpu.SemaphoreType.DMA((2, N_Z - 1)),
            pltpu.SemaphoreType.DMA((2, N_Z - 1)),
            pltpu.SemaphoreType.DMA((2, N_Z - 1)),
        ],
        compiler_params=pltpu.CompilerParams(collective_id=0),
    )(x)


# device time: 48717 ns/iter; 1.0012x vs baseline; 1.0012x over previous
import jax
import jax.numpy as jnp
from jax import lax
from jax.experimental import pallas as pl
from jax.experimental.pallas import tpu as pltpu

N_Z = 4
MESH = pl.DeviceIdType.MESH


def kernel(x):
    m_per, n = x.shape
    qrows = m_per // 4
    half = n // 2

    def body(x_ref, o_ref, zrs, zrr, zls, zlr,
             xs_, xr_, ys_, yr_, fxs, fxr, fys, fyr):
        cx = lax.axis_index("x")
        cy = lax.axis_index("y")
        z = lax.axis_index("z")
        q_me = 2 * cx + cy
        q_x = 2 * (1 - cx) + cy
        q_y = 2 * cx + (1 - cy)
        q_d = 2 * (1 - cx) + (1 - cy)
        xn = (1 - cx, cy, z)
        yn = (cx, 1 - cy, z)

        def cl(v):
            return jnp.clip(v, 0, N_Z - 1)

        def slab(j, q):
            return o_ref.at[pl.ds(j * m_per + q * qrows, qrows), :]

        def slab_half(j, q, hf):
            return o_ref.at[
                pl.ds(j * m_per + q * qrows, qrows), pl.ds(hf * half, half)
            ]

        bar = pltpu.get_barrier_semaphore()
        pl.semaphore_signal(bar, inc=1, device_id=xn, device_id_type=MESH)
        pl.semaphore_signal(bar, inc=1, device_id=yn, device_id_type=MESH)

        @pl.when(z >= 1)
        def _():
            pl.semaphore_signal(
                bar, inc=1, device_id=(cx, cy, cl(z - 1)), device_id_type=MESH
            )

        @pl.when(z <= 2)
        def _():
            pl.semaphore_signal(
                bar, inc=1, device_id=(cx, cy, cl(z + 1)), device_id_type=MESH
            )

        @pl.when(jnp.logical_and(z >= 1, z <= 2))
        def _():
            pl.semaphore_wait(bar, 4)

        @pl.when(jnp.logical_or(z == 0, z == 3))
        def _():
            pl.semaphore_wait(bar, 3)

        drains = []

        def z_send(h, right):
            if right:
                cond = jnp.logical_and(z >= h, z <= 2)
                j, tgt, ss, rs = cl(z - h), (cx, cy, cl(z + 1)), zrs, zrr
            else:
                cond = jnp.logical_and(z >= 1, z + h <= 3)
                j, tgt, ss, rs = cl(z + h), (cx, cy, cl(z - 1)), zls, zlr
            src = (
                x_ref.at[pl.ds(q_me * qrows, qrows), :]
                if h == 0
                else slab(j, q_me)
            )
            d = pltpu.make_async_remote_copy(
                src_ref=src, dst_ref=slab(j, q_me),
                send_sem=ss.at[h], recv_sem=rs.at[h],
                device_id=tgt, device_id_type=MESH,
            )

            @pl.when(cond)
            def _():
                d.start()

            drains.append((cond, d))

        def arr(h, right):
            if right:
                return z >= h + 1, cl(z - 1 - h)
            return z <= 2 - h, cl(z + 1 + h)

        def z_recv_wait(h, right):
            cond, j = arr(h, right)
            rs = zrr if right else zlr
            d = pltpu.make_async_remote_copy(
                src_ref=slab(j, q_me), dst_ref=slab(j, q_me),
                send_sem=rs.at[h], recv_sem=rs.at[h],
                device_id=(cx, cy, z), device_id_type=MESH,
            )

            @pl.when(cond)
            def _():
                d.wait_recv()

        def plane_direct_start(h, f):
            cond, j = arr(h, f)
            for sem_s, sem_r, tgt in ((xs_, xr_, xn), (ys_, yr_, yn)):
                d = pltpu.make_async_remote_copy(
                    src_ref=slab(j, q_me), dst_ref=slab(j, q_me),
                    send_sem=sem_s.at[int(f), h], recv_sem=sem_r.at[int(f), h],
                    device_id=tgt, device_id_type=MESH,
                )

                @pl.when(cond)
                def _():
                    d.start()

                drains.append((cond, d))

        def plane_recv_and_forward(h, f):
            cond, j = arr(h, f)
            dx = pltpu.make_async_remote_copy(
                src_ref=slab(j, q_x), dst_ref=slab(j, q_x),
                send_sem=xr_.at[int(f), h], recv_sem=xr_.at[int(f), h],
                device_id=(cx, cy, z), device_id_type=MESH,
            )
            dy = pltpu.make_async_remote_copy(
                src_ref=slab(j, q_y), dst_ref=slab(j, q_y),
                send_sem=yr_.at[int(f), h], recv_sem=yr_.at[int(f), h],
                device_id=(cx, cy, z), device_id_type=MESH,
            )
            fy = pltpu.make_async_remote_copy(
                src_ref=slab_half(j, q_x, 0), dst_ref=slab_half(j, q_x, 0),
                send_sem=fys.at[int(f), h], recv_sem=fyr.at[int(f), h],
                device_id=yn, device_id_type=MESH,
            )
            fx = pltpu.make_async_remote_copy(
                src_ref=slab_half(j, q_y, 1), dst_ref=slab_half(j, q_y, 1),
                send_sem=fxs.at[int(f), h], recv_sem=fxr.at[int(f), h],
                device_id=xn, device_id_type=MESH,
            )

            @pl.when(cond)
            def _():
                dx.wait_recv()
                fy.start()
                dy.wait_recv()
                fx.start()

            drains.append((cond, fy))
            drains.append((cond, fx))

        def fwd_recv_wait(h, f):
            cond, j = arr(h, f)
            da = pltpu.make_async_remote_copy(
                src_ref=slab_half(j, q_d, 0), dst_ref=slab_half(j, q_d, 0),
                send_sem=fys.at[int(f), h], recv_sem=fyr.at[int(f), h],
                device_id=(cx, cy, z), device_id_type=MESH,
            )
            db = pltpu.make_async_remote_copy(
                src_ref=slab_half(j, q_d, 1), dst_ref=slab_half(j, q_d, 1),
                send_sem=fxs.at[int(f), h], recv_sem=fxr.at[int(f), h],
                device_id=(cx, cy, z), device_id_type=MESH,
            )

            @pl.when(cond)
            def _():
                da.wait_recv()
                db.wait_recv()

        z_send(0, True)
        z_send(0, False)
        o_ref[pl.ds(z * m_per, m_per), :] = x_ref[...]
        for h in range(N_Z - 1):
            for f in (True, False):
                z_recv_wait(h, f)
                if h < N_Z - 2:
                    z_send(h + 1, f)
            for f in (True, False):
                plane_direct_start(h, f)
            for f in (True, False):
                plane_recv_and_forward(h, f)
        for h in range(N_Z - 1):
            for f in (True, False):
                fwd_recv_wait(h, f)
        for cond, d in drains:
            @pl.when(cond)
            def _():
                d.wait_send()

    out_shape = jax.ShapeDtypeStruct((N_Z * m_per, n), jnp.float32)
    return pl.pallas_call(
        body,
        out_shape=out_shape,
        in_specs=[pl.BlockSpec(memory_space=pltpu.VMEM)],
        out_specs=pl.BlockSpec(memory_space=pltpu.VMEM),
        scratch_shapes=[
            pltpu.SemaphoreType.DMA((N_Z - 1,)),
            pltpu.SemaphoreType.DMA((N_Z - 1,)),
            pltpu.SemaphoreType.DMA((N_Z - 1,)),
            pltpu.SemaphoreType.DMA((N_Z - 1,)),
            pltpu.SemaphoreType.DMA((2, N_Z - 1)),
            pltpu.SemaphoreType.DMA((2, N_Z - 1)),
            pltpu.SemaphoreType.DMA((2, N_Z - 1)),
            pltpu.SemaphoreType.DMA((2, N_Z - 1)),
            pltpu.SemaphoreType.DMA((2, N_Z - 1)),
            pltpu.SemaphoreType.DMA((2, N_Z - 1)),
            pltpu.SemaphoreType.DMA((2, N_Z - 1)),
            pltpu.SemaphoreType.DMA((2, N_Z - 1)),
        ],
        compiler_params=pltpu.CompilerParams(collective_id=0),
    )(x)
